# baseline (speedup 1.0000x reference)
import jax
import jax.numpy as jnp
from jax import lax
from jax.experimental import pallas as pl
from jax.experimental.pallas import tpu as pltpu
from jax.experimental.pallas import tpu_sc as plsc

B, R, G, D = 4, 8, 10000, 128
NC, NS, L = 2, 16, 16
NW = NC * NS
CG = 80
CHUNKS_PER_B = G // CG
CHUNKS = B * CHUNKS_PER_B
BASE_CHUNKS = CHUNKS // NW
EXTRA = CHUNKS % NW
NGRP = CG // L


def _body(x_hbm, out_hbm, buf, outbuf, sem):
    wid = lax.axis_index("s") * NC + lax.axis_index("c")
    n_chunks = jnp.where(wid < EXTRA, BASE_CHUNKS + 1, BASE_CHUNKS)
    lanes = lax.iota(jnp.int32, L)

    def chunk_body(ci, _):
        c = wid + ci * NW
        b = c // CHUNKS_PER_B
        g0 = (c % CHUNKS_PER_B) * CG
        cps = [
            pltpu.async_copy(x_hbm.at[b, r, pl.ds(g0, CG)], buf.at[r], sem)
            for r in range(R)
        ]
        for cp in cps:
            cp.wait()

        for grp in range(NGRP):
            g_loc = jnp.full((L,), grp * L, jnp.int32) + lanes
            best = jnp.full((L,), -jnp.inf, jnp.float32)
            best_r = jnp.zeros((L,), jnp.int32)
            for r in range(R):
                r_vec = jnp.full((L,), r, jnp.int32)

                def norm_step(d, acc, r_vec=r_vec, g_loc=g_loc):
                    d_vec = jnp.broadcast_to(d, (L,)).astype(jnp.int32)
                    v = plsc.load_gather(buf, [r_vec, g_loc, d_vec])
                    return acc + v * v

                norm = lax.fori_loop(0, D, norm_step,
                                     jnp.zeros((L,), jnp.float32))
                upd = norm > best
                best = jnp.where(upd, norm, best)
                best_r = jnp.where(upd, r_vec, best_r)

            def copy_step(d, _, best_r=best_r, g_loc=g_loc):
                d_vec = jnp.broadcast_to(d, (L,)).astype(jnp.int32)
                v = plsc.load_gather(buf, [best_r, g_loc, d_vec])
                plsc.store_scatter(outbuf, [g_loc, d_vec], v)
                return 0

            lax.fori_loop(0, D, copy_step, 0)

        pltpu.sync_copy(outbuf, out_hbm.at[b, pl.ds(g0, CG)])
        return 0

    lax.fori_loop(0, n_chunks, chunk_body, 0)


@jax.jit
def kernel(inputs):
    mesh = plsc.VectorSubcoreMesh(core_axis_name="c", subcore_axis_name="s")
    f = pl.kernel(
        _body,
        out_type=jax.ShapeDtypeStruct((B, G, D), jnp.float32),
        mesh=mesh,
        scratch_types=[
            pltpu.VMEM((R, CG, D), jnp.float32),
            pltpu.VMEM((CG, D), jnp.float32),
            pltpu.SemaphoreType.DMA,
        ],
        compiler_params=pltpu.CompilerParams(
            use_tc_tiling_on_sc=False, needs_layout_passes=False
        ),
    )
    return f(inputs)


# speedup vs baseline: 1.9383x; 1.9383x over previous
import jax
import jax.numpy as jnp
from jax import lax
from jax.experimental import pallas as pl
from jax.experimental.pallas import tpu as pltpu
from jax.experimental.pallas import tpu_sc as plsc

B, R, G, D = 4, 8, 10000, 128
DP = D + 1
NC, NS, L = 2, 16, 16
NW = NC * NS
CG = 80
CHUNKS_PER_B = G // CG
CHUNKS = B * CHUNKS_PER_B
BASE_CHUNKS = CHUNKS // NW
EXTRA = CHUNKS % NW
NGRP = CG // L


def _body(x_hbm, out_hbm, buf, outbuf, sem):
    wid = lax.axis_index("s") * NC + lax.axis_index("c")
    n_chunks = jnp.where(wid < EXTRA, BASE_CHUNKS + 1, BASE_CHUNKS)
    lanes = lax.iota(jnp.int32, L)

    def chunk_body(ci, _):
        c = wid + ci * NW
        b = c // CHUNKS_PER_B
        g0 = (c % CHUNKS_PER_B) * CG
        cps = [
            pltpu.async_copy(
                x_hbm.at[b, r, pl.ds(g0, CG)],
                buf.at[r, :, pl.ds(0, D)],
                sem,
            )
            for r in range(R)
        ]
        for cp in cps:
            cp.wait()

        for grp in range(NGRP):
            g_loc = jnp.full((L,), grp * L, jnp.int32) + lanes
            best = jnp.full((L,), -jnp.inf, jnp.float32)
            best_r = jnp.zeros((L,), jnp.int32)
            for r in range(R):
                r_vec = jnp.full((L,), r, jnp.int32)

                def norm_step(d, acc, r_vec=r_vec, g_loc=g_loc):
                    d_vec = jnp.broadcast_to(d, (L,)).astype(jnp.int32)
                    v = plsc.load_gather(buf, [r_vec, g_loc, d_vec])
                    return acc + v * v

                norm = lax.fori_loop(0, D, norm_step,
                                     jnp.zeros((L,), jnp.float32))
                upd = norm > best
                best = jnp.where(upd, norm, best)
                best_r = jnp.where(upd, r_vec, best_r)

            def copy_step(d, _, best_r=best_r, g_loc=g_loc):
                d_vec = jnp.broadcast_to(d, (L,)).astype(jnp.int32)
                v = plsc.load_gather(buf, [best_r, g_loc, d_vec])
                plsc.store_scatter(outbuf, [g_loc, d_vec], v)
                return 0

            lax.fori_loop(0, D, copy_step, 0)

        pltpu.sync_copy(outbuf.at[:, pl.ds(0, D)], out_hbm.at[b, pl.ds(g0, CG)])
        return 0

    lax.fori_loop(0, n_chunks, chunk_body, 0)


@jax.jit
def kernel(inputs):
    mesh = plsc.VectorSubcoreMesh(core_axis_name="c", subcore_axis_name="s")
    f = pl.kernel(
        _body,
        out_type=jax.ShapeDtypeStruct((B, G, D), jnp.float32),
        mesh=mesh,
        scratch_types=[
            pltpu.VMEM((R, CG, DP), jnp.float32),
            pltpu.VMEM((CG, DP), jnp.float32),
            pltpu.SemaphoreType.DMA,
        ],
        compiler_params=pltpu.CompilerParams(
            use_tc_tiling_on_sc=False, needs_layout_passes=False
        ),
    )
    return f(inputs)


# speedup vs baseline: 4.4132x; 2.2768x over previous
import jax
import jax.numpy as jnp
import numpy as np
from jax import lax
from jax.experimental import pallas as pl
from jax.experimental.pallas import tpu as pltpu
from jax.experimental.pallas import tpu_sc as plsc

B, R, G, D = 4, 8, 10000, 128
DP = D + 1
NC, NS, L = 2, 16, 16
NW = NC * NS
CG = 40
CHUNKS_PER_B = G // CG
CHUNKS = B * CHUNKS_PER_B
BASE_CHUNKS = CHUNKS // NW
EXTRA = CHUNKS % NW
PAIRS = (BASE_CHUNKS + 2) // 2
GRP_OFF = (0, 16, CG - L)
UNROLL = 4


def _chunk_coords(c):
    b = c // CHUNKS_PER_B
    g0 = (c % CHUNKS_PER_B) * CG
    return b, g0


def _issue_in(x_hbm, c, buf, sem):
    b, g0 = _chunk_coords(c)
    for r in range(R):
        pltpu.async_copy(
            x_hbm.at[b, r, pl.ds(g0, CG)], buf.at[r, :, pl.ds(0, D)], sem
        )


def _wait_in(x_hbm, c, buf, sem):
    b, g0 = _chunk_coords(c)
    for r in range(R):
        pltpu.make_async_copy(
            x_hbm.at[b, r, pl.ds(g0, CG)], buf.at[r, :, pl.ds(0, D)], sem
        ).wait()


def _out_copy(out_hbm, c, outbuf, sem):
    b, g0 = _chunk_coords(c)
    return pltpu.make_async_copy(
        outbuf.at[:, pl.ds(0, D)], out_hbm.at[b, pl.ds(g0, CG)], sem
    )


def _compute_chunk(buf, outbuf):
    for off in GRP_OFF:
        g_loc = lax.iota(jnp.int32, L) + off
        r_vecs = [jnp.full((L,), r, jnp.int32) for r in range(R)]

        def norm_step(d, accs, g_loc=g_loc, r_vecs=r_vecs):
            d_vec = jnp.broadcast_to(d, (L,)).astype(jnp.int32)
            out = []
            for r in range(R):
                v = plsc.load_gather(buf, [r_vecs[r], g_loc, d_vec])
                out.append(accs[r] + v * v)
            return tuple(out)

        accs = lax.fori_loop(
            0, D, norm_step,
            tuple(jnp.zeros((L,), jnp.float32) for _ in range(R)),
            unroll=UNROLL,
        )

        best = accs[0]
        best_r = jnp.zeros((L,), jnp.int32)
        for r in range(1, R):
            upd = accs[r] > best
            best = jnp.where(upd, accs[r], best)
            best_r = jnp.where(upd, r_vecs[r], best_r)

        def copy_step(d, carry, g_loc=g_loc, best_r=best_r):
            d_vec = jnp.broadcast_to(d, (L,)).astype(jnp.int32)
            v = plsc.load_gather(buf, [best_r, g_loc, d_vec])
            plsc.store_scatter(outbuf, [g_loc, d_vec], v)
            return carry

        lax.fori_loop(0, D, copy_step, 0, unroll=UNROLL)


def _body(x_hbm, out_hbm, buf0, buf1, outbuf0, outbuf1,
          sem_i0, sem_i1, sem_o0, sem_o1):
    wid = lax.axis_index("s") * NC + lax.axis_index("c")
    n = jnp.where(wid < EXTRA, BASE_CHUNKS + 1, BASE_CHUNKS)

    _issue_in(x_hbm, wid, buf0, sem_i0)

    def pair_body(k, carry):
        c0 = wid + (2 * k) * NW
        c1 = wid + (2 * k + 1) * NW

        _wait_in(x_hbm, c0, buf0, sem_i0)

        @pl.when(2 * k + 1 < n)
        def _():
            _issue_in(x_hbm, c1, buf1, sem_i1)

        @pl.when(k >= 1)
        def _():
            _out_copy(out_hbm, c0 - 2 * NW, outbuf0, sem_o0).wait()

        _compute_chunk(buf0, outbuf0)
        _out_copy(out_hbm, c0, outbuf0, sem_o0).start()

        @pl.when(2 * k + 1 < n)
        def _():
            _wait_in(x_hbm, c1, buf1, sem_i1)

            @pl.when(2 * k + 2 < n)
            def _():
                _issue_in(x_hbm, c1 + NW, buf0, sem_i0)

            @pl.when(k >= 1)
            def _():
                _out_copy(out_hbm, c1 - 2 * NW, outbuf1, sem_o1).wait()

            _compute_chunk(buf1, outbuf1)
            _out_copy(out_hbm, c1, outbuf1, sem_o1).start()

        return carry

    lax.fori_loop(0, PAIRS, pair_body, 0)

    _out_copy(out_hbm, wid, outbuf0, sem_o0).wait()
    _out_copy(out_hbm, wid, outbuf1, sem_o1).wait()


@jax.jit
def kernel(inputs):
    mesh = plsc.VectorSubcoreMesh(core_axis_name="c", subcore_axis_name="s")
    f = pl.kernel(
        _body,
        out_type=jax.ShapeDtypeStruct((B, G, D), jnp.float32),
        mesh=mesh,
        scratch_types=[
            pltpu.VMEM((R, CG, DP), jnp.float32),
            pltpu.VMEM((R, CG, DP), jnp.float32),
            pltpu.VMEM((CG, DP), jnp.float32),
            pltpu.VMEM((CG, DP), jnp.float32),
            pltpu.SemaphoreType.DMA,
            pltpu.SemaphoreType.DMA,
            pltpu.SemaphoreType.DMA,
            pltpu.SemaphoreType.DMA,
        ],
        compiler_params=pltpu.CompilerParams(
            use_tc_tiling_on_sc=False, needs_layout_passes=False
        ),
    )
    return f(inputs)
